# baseline (speedup 1.0000x reference)
import functools

import jax
import jax.numpy as jnp
from jax import lax
from jax.experimental import pallas as pl
from jax.experimental.pallas import tpu as pltpu
from jax.experimental.pallas import tpu_sc as plsc

N = 10000
D = 64
E = 160000
E2 = E + N
NFEAT = 256
K = 10
ALPHA = 0.1

NT = 16
B = 128
NB = 85
EPT = NB * B
EPAD = NT * EPT
CH = 640
SUB = 80
NSUB = CH // SUB
NPAD = NT * CH


def _mlp_body(x_ref, w1_ref, b1_ref, w2_ref, b2_ref, o_ref):
    h = lax.dot_general(x_ref[...], w1_ref[...], (((1,), (1,)), ((), ())),
                        preferred_element_type=jnp.float32)
    h = jnp.maximum(h + b1_ref[...], 0.0)
    o = lax.dot_general(h, w2_ref[...], (((1,), (1,)), ((), ())),
                        preferred_element_type=jnp.float32)
    o_ref[...] = o + b2_ref[...]


def _mlp(x, W1, b1, W2, b2):
    R = 640
    return pl.pallas_call(
        _mlp_body,
        grid=(NPAD // R,),
        in_specs=[
            pl.BlockSpec((R, NFEAT), lambda i: (i, 0)),
            pl.BlockSpec((D, NFEAT), lambda i: (0, 0)),
            pl.BlockSpec((1, D), lambda i: (0, 0)),
            pl.BlockSpec((D, D), lambda i: (0, 0)),
            pl.BlockSpec((1, D), lambda i: (0, 0)),
        ],
        out_specs=pl.BlockSpec((R, D), lambda i: (i, 0)),
        out_shape=jax.ShapeDtypeStruct((NPAD, D), jnp.float32),
    )(x, W1, b1.reshape(1, D), W2, b2.reshape(1, D))


def _prop_body(h0_hbm, src_hbm, dst_hbm, out_hbm, g_hbm,
               acc, deg, srcv, dstv,
               rows0, rows1, rows2,
               ab0, ab1, hb0, hb1, gb0, gb1, zb,
               ones, zeros1, dbuf, dinv, asc, bsc,
               gsem, ssem, rsem_a, rsem_h, wsem_g, wsem_z):
    s = lax.axis_index("s")

    pltpu.sync_copy(src_hbm.at[pl.ds(s * NB, NB)], srcv)
    pltpu.sync_copy(dst_hbm.at[pl.ds(s * NB, NB)], dstv)

    def _initv(t, _):
        sl = pl.ds(t * 16, 16)
        ones[sl] = jnp.ones((16,), jnp.float32)
        zeros1[sl] = jnp.zeros((16,), jnp.float32)
        dbuf[sl] = jnp.ones((16,), jnp.float32)
        return 0
    lax.fori_loop(0, CH // 16, _initv, 0)

    def _initz(i, _):
        for q in range(D // 16):
            zb[i, pl.ds(q * 16, 16)] = jnp.zeros((16,), jnp.float32)
        return 0
    lax.fori_loop(0, SUB, _initz, 0)

    pltpu.sync_copy(zeros1, deg.at[pl.ds(s * CH, CH)])
    plsc.subcore_barrier()

    def _degb(j, _):
        @pl.when(j >= 4)
        def _():
            pltpu.make_async_copy(ones.at[pl.ds(0, B)], deg.at[dstv.at[j - 4]],
                                  ssem.at[j % 4]).wait()
        pltpu.make_async_copy(ones.at[pl.ds(0, B)], deg.at[dstv.at[j]],
                              ssem.at[j % 4]).start(add=True)
        return 0
    lax.fori_loop(0, NB, _degb, 0)
    for r in range(4):
        j = NB - 4 + r
        pltpu.make_async_copy(ones.at[pl.ds(0, B)], deg.at[dstv.at[j]],
                              ssem.at[j % 4]).wait()
    plsc.subcore_barrier()

    pltpu.sync_copy(deg.at[pl.ds(s * CH, CH)], dbuf)

    def _newton(t, _):
        sl = pl.ds(t * 16, 16)
        x = dbuf[sl]
        xi = lax.bitcast_convert_type(x, jnp.int32)
        yi = jnp.int32(0x5F3759DF) - lax.shift_right_logical(xi, 1)
        y = lax.bitcast_convert_type(yi, jnp.float32)
        for _ in range(3):
            y = y * (1.5 - 0.5 * x * y * y)
        dinv[sl] = y
        asc[sl] = (1.0 - ALPHA) * y * y
        bsc[sl] = ALPHA * y
        return 0
    lax.fori_loop(0, CH // 16, _newton, 0)

    def _g0sub(c5, _):
        pltpu.sync_copy(h0_hbm.at[pl.ds(s * CH + c5 * SUB, SUB)], hb0)

        def _g0grp(gi, _):
            lg = c5 * SUB + gi * 16
            d16 = dinv[pl.ds(lg, 16)]
            for i in range(16):
                dv = d16[i]
                for q in range(D // 16):
                    sl = pl.ds(q * 16, 16)
                    gb0[gi * 16 + i, sl] = dv * hb0[gi * 16 + i, sl]
            return 0
        lax.fori_loop(0, SUB // 16, _g0grp, 0)
        pltpu.sync_copy(gb0, g_hbm.at[pl.ds(s * CH + c5 * SUB, SUB)])
        pltpu.sync_copy(zb, acc.at[pl.ds(s * CH + c5 * SUB, SUB)])
        return 0
    lax.fori_loop(0, NSUB, _g0sub, 0)
    plsc.subcore_barrier()

    rbufs = (rows0, rows1, rows2)

    def _estep(j, b, guarded):
        b2 = (b + 1) % 3

        def _w():
            pltpu.make_async_copy(rbufs[b2], acc.at[dstv.at[j - 2]],
                                  ssem.at[b2]).wait()

        def _g():
            pltpu.make_async_copy(g_hbm.at[srcv.at[j + 1]], rbufs[b2],
                                  gsem.at[b2]).start()
        if guarded:
            pl.when(j >= 2)(_w)
            _g()
        else:
            _w()
        pltpu.make_async_copy(g_hbm.at[srcv.at[j]], rbufs[b],
                              gsem.at[b]).wait()
        pltpu.make_async_copy(rbufs[b], acc.at[dstv.at[j]],
                              ssem.at[b]).start(add=True)

    def _edge_phase():
        pltpu.make_async_copy(g_hbm.at[srcv.at[0]], rbufs[0], gsem.at[0]).start()

        def _eb(t, _):
            for b in range(3):
                _estep(t * 3 + b, b, True)
            return 0
        lax.fori_loop(0, (NB - 1) // 3, _eb, 0)
        _estep(NB - 1, (NB - 1) % 3, False)
        for j in (NB - 2, NB - 1):
            pltpu.make_async_copy(rbufs[j % 3], acc.at[dstv.at[j]],
                                  ssem.at[j % 3]).wait()

    abufs = (ab0, ab1)
    hbufs = (hb0, hb1)
    gbufs = (gb0, gb1)

    def _u_reads(c5, p):
        n0 = s * CH + c5 * SUB
        pltpu.make_async_copy(acc.at[pl.ds(n0, SUB)], abufs[p], rsem_a.at[p]).start()
        pltpu.make_async_copy(h0_hbm.at[pl.ds(n0, SUB)], hbufs[p], rsem_h.at[p]).start()

    def _u_step(c5, p, last):
        n0 = s * CH + c5 * SUB
        ab, hb, gb = abufs[p], hbufs[p], gbufs[p]

        @pl.when(c5 + 1 < NSUB)
        def _():
            _u_reads(c5 + 1, 1 - p)

        pltpu.make_async_copy(acc.at[pl.ds(n0, SUB)], ab, rsem_a.at[p]).wait()
        pltpu.make_async_copy(h0_hbm.at[pl.ds(n0, SUB)], hb, rsem_h.at[p]).wait()

        @pl.when(c5 >= 2)
        def _():
            nprev = n0 - 2 * SUB
            dprev = out_hbm if last else g_hbm
            pltpu.make_async_copy(gb, dprev.at[pl.ds(nprev, SUB)], wsem_g.at[p]).wait()
            pltpu.make_async_copy(zb, acc.at[pl.ds(nprev, SUB)], wsem_z.at[p]).wait()

        def _ugrp(gi, _):
            lg = c5 * SUB + gi * 16
            if last:
                a16 = (1.0 - ALPHA) * dinv[pl.ds(lg, 16)]
                b16 = jnp.full((16,), ALPHA, jnp.float32)
            else:
                a16 = asc[pl.ds(lg, 16)]
                b16 = bsc[pl.ds(lg, 16)]
            for i in range(16):
                a_s = a16[i]
                b_s = b16[i]
                r = gi * 16 + i
                for q in range(D // 16):
                    sl = pl.ds(q * 16, 16)
                    gb[r, sl] = a_s * ab[r, sl] + b_s * hb[r, sl]
            return 0
        lax.fori_loop(0, SUB // 16, _ugrp, 0)
        dref = out_hbm if last else g_hbm
        pltpu.make_async_copy(gb, dref.at[pl.ds(n0, SUB)], wsem_g.at[p]).start()
        pltpu.make_async_copy(zb, acc.at[pl.ds(n0, SUB)], wsem_z.at[p]).start()

    def _update_phase(last):
        _u_reads(0, 0)

        def _ut(t, _):
            for pb in range(2):
                _u_step(t * 2 + pb, pb, last)
            return 0
        lax.fori_loop(0, NSUB // 2, _ut, 0)
        dref = out_hbm if last else g_hbm
        for c5 in (NSUB - 2, NSUB - 1):
            p = c5 % 2
            n0 = s * CH + c5 * SUB
            pltpu.make_async_copy(gbufs[p], dref.at[pl.ds(n0, SUB)], wsem_g.at[p]).wait()
            pltpu.make_async_copy(zb, acc.at[pl.ds(n0, SUB)], wsem_z.at[p]).wait()

    def _iteration(last):
        _edge_phase()
        plsc.subcore_barrier()
        _update_phase(last)
        plsc.subcore_barrier()

    def _kb(k, _):
        _iteration(False)
        return 0
    lax.fori_loop(0, K - 1, _kb, 0)
    _iteration(True)


_prop = functools.partial(
    pl.kernel,
    out_type=(jax.ShapeDtypeStruct((NPAD, D), jnp.float32),
              jax.ShapeDtypeStruct((NPAD, D), jnp.float32)),
    mesh=plsc.VectorSubcoreMesh(core_axis_name="c", subcore_axis_name="s",
                                num_cores=1, num_subcores=NT),
    compiler_params=pltpu.CompilerParams(use_tc_tiling_on_sc=False),
    scratch_types=[
        pltpu.VMEM_SHARED((NPAD, D), jnp.float32),
        pltpu.VMEM_SHARED((NPAD,), jnp.float32),
        pltpu.VMEM((NB, B), jnp.int32),
        pltpu.VMEM((NB, B), jnp.int32),
        pltpu.VMEM((B, D), jnp.float32),
        pltpu.VMEM((B, D), jnp.float32),
        pltpu.VMEM((B, D), jnp.float32),
        pltpu.VMEM((SUB, D), jnp.float32),
        pltpu.VMEM((SUB, D), jnp.float32),
        pltpu.VMEM((SUB, D), jnp.float32),
        pltpu.VMEM((SUB, D), jnp.float32),
        pltpu.VMEM((SUB, D), jnp.float32),
        pltpu.VMEM((SUB, D), jnp.float32),
        pltpu.VMEM((SUB, D), jnp.float32),
        pltpu.VMEM((CH,), jnp.float32),
        pltpu.VMEM((CH,), jnp.float32),
        pltpu.VMEM((CH,), jnp.float32),
        pltpu.VMEM((CH,), jnp.float32),
        pltpu.VMEM((CH,), jnp.float32),
        pltpu.VMEM((CH,), jnp.float32),
        pltpu.SemaphoreType.DMA((3,)),
        pltpu.SemaphoreType.DMA((4,)),
        pltpu.SemaphoreType.DMA((2,)),
        pltpu.SemaphoreType.DMA((2,)),
        pltpu.SemaphoreType.DMA((2,)),
        pltpu.SemaphoreType.DMA((2,)),
    ],
)(_prop_body)


def kernel(x, edge_index, W1, b1, W2, b2):
    h0p = _mlp(x, W1, b1, W2, b2)
    loop = jnp.arange(N, dtype=jnp.int32)
    pad = N + (jnp.arange(EPAD - E2, dtype=jnp.int32) % (NPAD - N))
    src = jnp.concatenate([edge_index[0].astype(jnp.int32), loop, pad])
    dst = jnp.concatenate([edge_index[1].astype(jnp.int32), loop, pad])
    out, _ = _prop(h0p, src.reshape(NT * NB, B), dst.reshape(NT * NB, B))
    return out[:N]


# speedup vs baseline: 1.0096x; 1.0096x over previous
import functools

import jax
import jax.numpy as jnp
from jax import lax
from jax.experimental import pallas as pl
from jax.experimental.pallas import tpu as pltpu
from jax.experimental.pallas import tpu_sc as plsc

N = 10000
D = 64
E = 160000
E2 = E + N
NFEAT = 256
K = 10
ALPHA = 0.1

NT = 16
B = 128
NB = 85
EPT = NB * B
EPAD = NT * EPT
CH = 640
SUB = 80
NSUB = CH // SUB
NPAD = NT * CH


def _mlp_body(x_ref, w1_ref, b1_ref, w2_ref, b2_ref, o_ref):
    h = lax.dot_general(x_ref[...], w1_ref[...], (((1,), (1,)), ((), ())),
                        preferred_element_type=jnp.float32)
    h = jnp.maximum(h + b1_ref[...], 0.0)
    o = lax.dot_general(h, w2_ref[...], (((1,), (1,)), ((), ())),
                        preferred_element_type=jnp.float32)
    o_ref[...] = o + b2_ref[...]


def _mlp(x, W1, b1, W2, b2):
    R = 1000
    return pl.pallas_call(
        _mlp_body,
        grid=(N // R,),
        in_specs=[
            pl.BlockSpec((R, NFEAT), lambda i: (i, 0)),
            pl.BlockSpec((D, NFEAT), lambda i: (0, 0)),
            pl.BlockSpec((1, D), lambda i: (0, 0)),
            pl.BlockSpec((D, D), lambda i: (0, 0)),
            pl.BlockSpec((1, D), lambda i: (0, 0)),
        ],
        out_specs=pl.BlockSpec((R, D), lambda i: (i, 0)),
        out_shape=jax.ShapeDtypeStruct((N, D), jnp.float32),
    )(x, W1, b1.reshape(1, D), W2, b2.reshape(1, D))


def _prop_body(h0_hbm, src_hbm, dst_hbm, out_hbm, g_hbm,
               acc, deg, srcv, dstv,
               rows0, rows1, rows2,
               ab0, ab1, hb0, hb1, gb0, gb1, zb,
               ones, zeros1, dbuf, dinv, asc, bsc,
               gsem, ssem, rsem_a, rsem_h, wsem_g, wsem_z):
    s = lax.axis_index("s")

    pltpu.sync_copy(src_hbm.at[pl.ds(s * NB, NB)], srcv)
    pltpu.sync_copy(dst_hbm.at[pl.ds(s * NB, NB)], dstv)

    def _initv(t, _):
        sl = pl.ds(t * 16, 16)
        ones[sl] = jnp.ones((16,), jnp.float32)
        zeros1[sl] = jnp.zeros((16,), jnp.float32)
        dbuf[sl] = jnp.ones((16,), jnp.float32)
        return 0
    lax.fori_loop(0, CH // 16, _initv, 0)

    def _initz(i, _):
        for q in range(D // 16):
            zb[i, pl.ds(q * 16, 16)] = jnp.zeros((16,), jnp.float32)
        return 0
    lax.fori_loop(0, SUB, _initz, 0)

    pltpu.sync_copy(zeros1, deg.at[pl.ds(s * CH, CH)])
    plsc.subcore_barrier()

    def _degb(j, _):
        @pl.when(j >= 4)
        def _():
            pltpu.make_async_copy(ones.at[pl.ds(0, B)], deg.at[dstv.at[j - 4]],
                                  ssem.at[j % 4]).wait()
        pltpu.make_async_copy(ones.at[pl.ds(0, B)], deg.at[dstv.at[j]],
                              ssem.at[j % 4]).start(add=True)
        return 0
    lax.fori_loop(0, NB, _degb, 0)
    for r in range(4):
        j = NB - 4 + r
        pltpu.make_async_copy(ones.at[pl.ds(0, B)], deg.at[dstv.at[j]],
                              ssem.at[j % 4]).wait()
    plsc.subcore_barrier()

    pltpu.sync_copy(deg.at[pl.ds(s * CH, CH)], dbuf)

    def _newton(t, _):
        sl = pl.ds(t * 16, 16)
        x = dbuf[sl]
        xi = lax.bitcast_convert_type(x, jnp.int32)
        yi = jnp.int32(0x5F3759DF) - lax.shift_right_logical(xi, 1)
        y = lax.bitcast_convert_type(yi, jnp.float32)
        for _ in range(3):
            y = y * (1.5 - 0.5 * x * y * y)
        dinv[sl] = y
        asc[sl] = (1.0 - ALPHA) * y * y
        bsc[sl] = ALPHA * y
        return 0
    lax.fori_loop(0, CH // 16, _newton, 0)

    def _g0sub(c5, _):
        pltpu.sync_copy(h0_hbm.at[pl.ds(s * CH + c5 * SUB, SUB)], hb0)

        def _g0grp(gi, _):
            lg = c5 * SUB + gi * 16
            d16 = dinv[pl.ds(lg, 16)]
            for i in range(16):
                dv = d16[i]
                for q in range(D // 16):
                    sl = pl.ds(q * 16, 16)
                    gb0[gi * 16 + i, sl] = dv * hb0[gi * 16 + i, sl]
            return 0
        lax.fori_loop(0, SUB // 16, _g0grp, 0)
        pltpu.sync_copy(gb0, g_hbm.at[pl.ds(s * CH + c5 * SUB, SUB)])
        pltpu.sync_copy(zb, acc.at[pl.ds(s * CH + c5 * SUB, SUB)])
        return 0
    lax.fori_loop(0, NSUB, _g0sub, 0)
    plsc.subcore_barrier()

    rbufs = (rows0, rows1, rows2)

    def _estep(j, b, guarded):
        b2 = (b + 1) % 3

        def _w():
            pltpu.make_async_copy(rbufs[b2], acc.at[dstv.at[j - 2]],
                                  ssem.at[b2]).wait()

        def _g():
            pltpu.make_async_copy(g_hbm.at[srcv.at[j + 1]], rbufs[b2],
                                  gsem.at[b2]).start()
        if guarded:
            pl.when(j >= 2)(_w)
            _g()
        else:
            _w()
        pltpu.make_async_copy(g_hbm.at[srcv.at[j]], rbufs[b],
                              gsem.at[b]).wait()
        pltpu.make_async_copy(rbufs[b], acc.at[dstv.at[j]],
                              ssem.at[b]).start(add=True)

    def _edge_phase():
        pltpu.make_async_copy(g_hbm.at[srcv.at[0]], rbufs[0], gsem.at[0]).start()

        def _eb(t, _):
            for b in range(3):
                _estep(t * 3 + b, b, True)
            return 0
        lax.fori_loop(0, (NB - 1) // 3, _eb, 0)
        _estep(NB - 1, (NB - 1) % 3, False)
        for j in (NB - 2, NB - 1):
            pltpu.make_async_copy(rbufs[j % 3], acc.at[dstv.at[j]],
                                  ssem.at[j % 3]).wait()

    abufs = (ab0, ab1)
    hbufs = (hb0, hb1)
    gbufs = (gb0, gb1)

    def _u_reads(c5, p):
        n0 = s * CH + c5 * SUB
        pltpu.make_async_copy(acc.at[pl.ds(n0, SUB)], abufs[p], rsem_a.at[p]).start()
        pltpu.make_async_copy(h0_hbm.at[pl.ds(n0, SUB)], hbufs[p], rsem_h.at[p]).start()

    def _u_step(c5, p, last):
        n0 = s * CH + c5 * SUB
        ab, hb, gb = abufs[p], hbufs[p], gbufs[p]

        @pl.when(c5 + 1 < NSUB)
        def _():
            _u_reads(c5 + 1, 1 - p)

        pltpu.make_async_copy(acc.at[pl.ds(n0, SUB)], ab, rsem_a.at[p]).wait()
        pltpu.make_async_copy(h0_hbm.at[pl.ds(n0, SUB)], hb, rsem_h.at[p]).wait()

        @pl.when(c5 >= 2)
        def _():
            nprev = n0 - 2 * SUB
            dprev = out_hbm if last else g_hbm
            pltpu.make_async_copy(gb, dprev.at[pl.ds(nprev, SUB)], wsem_g.at[p]).wait()
            pltpu.make_async_copy(zb, acc.at[pl.ds(nprev, SUB)], wsem_z.at[p]).wait()

        def _ugrp(gi, _):
            lg = c5 * SUB + gi * 16
            if last:
                a16 = (1.0 - ALPHA) * dinv[pl.ds(lg, 16)]
                b16 = jnp.full((16,), ALPHA, jnp.float32)
            else:
                a16 = asc[pl.ds(lg, 16)]
                b16 = bsc[pl.ds(lg, 16)]
            for i in range(16):
                a_s = a16[i]
                b_s = b16[i]
                r = gi * 16 + i
                for q in range(D // 16):
                    sl = pl.ds(q * 16, 16)
                    gb[r, sl] = a_s * ab[r, sl] + b_s * hb[r, sl]
            return 0
        lax.fori_loop(0, SUB // 16, _ugrp, 0)
        dref = out_hbm if last else g_hbm
        pltpu.make_async_copy(gb, dref.at[pl.ds(n0, SUB)], wsem_g.at[p]).start()
        pltpu.make_async_copy(zb, acc.at[pl.ds(n0, SUB)], wsem_z.at[p]).start()

    def _update_phase(last):
        _u_reads(0, 0)

        def _ut(t, _):
            for pb in range(2):
                _u_step(t * 2 + pb, pb, last)
            return 0
        lax.fori_loop(0, NSUB // 2, _ut, 0)
        dref = out_hbm if last else g_hbm
        for c5 in (NSUB - 2, NSUB - 1):
            p = c5 % 2
            n0 = s * CH + c5 * SUB
            pltpu.make_async_copy(gbufs[p], dref.at[pl.ds(n0, SUB)], wsem_g.at[p]).wait()
            pltpu.make_async_copy(zb, acc.at[pl.ds(n0, SUB)], wsem_z.at[p]).wait()

    def _iteration(last):
        _edge_phase()
        plsc.subcore_barrier()
        _update_phase(last)
        plsc.subcore_barrier()

    def _kb(k, _):
        _iteration(False)
        return 0
    lax.fori_loop(0, K - 1, _kb, 0)
    _iteration(True)


_prop = functools.partial(
    pl.kernel,
    out_type=(jax.ShapeDtypeStruct((NPAD, D), jnp.float32),
              jax.ShapeDtypeStruct((NPAD, D), jnp.float32)),
    mesh=plsc.VectorSubcoreMesh(core_axis_name="c", subcore_axis_name="s",
                                num_cores=1, num_subcores=NT),
    compiler_params=pltpu.CompilerParams(use_tc_tiling_on_sc=False),
    scratch_types=[
        pltpu.VMEM_SHARED((NPAD, D), jnp.float32),
        pltpu.VMEM_SHARED((NPAD,), jnp.float32),
        pltpu.VMEM((NB, B), jnp.int32),
        pltpu.VMEM((NB, B), jnp.int32),
        pltpu.VMEM((B, D), jnp.float32),
        pltpu.VMEM((B, D), jnp.float32),
        pltpu.VMEM((B, D), jnp.float32),
        pltpu.VMEM((SUB, D), jnp.float32),
        pltpu.VMEM((SUB, D), jnp.float32),
        pltpu.VMEM((SUB, D), jnp.float32),
        pltpu.VMEM((SUB, D), jnp.float32),
        pltpu.VMEM((SUB, D), jnp.float32),
        pltpu.VMEM((SUB, D), jnp.float32),
        pltpu.VMEM((SUB, D), jnp.float32),
        pltpu.VMEM((CH,), jnp.float32),
        pltpu.VMEM((CH,), jnp.float32),
        pltpu.VMEM((CH,), jnp.float32),
        pltpu.VMEM((CH,), jnp.float32),
        pltpu.VMEM((CH,), jnp.float32),
        pltpu.VMEM((CH,), jnp.float32),
        pltpu.SemaphoreType.DMA((3,)),
        pltpu.SemaphoreType.DMA((4,)),
        pltpu.SemaphoreType.DMA((2,)),
        pltpu.SemaphoreType.DMA((2,)),
        pltpu.SemaphoreType.DMA((2,)),
        pltpu.SemaphoreType.DMA((2,)),
    ],
)(_prop_body)


def kernel(x, edge_index, W1, b1, W2, b2):
    h0 = _mlp(x, W1, b1, W2, b2)
    h0p = jnp.concatenate([h0, jnp.zeros((NPAD - N, D), jnp.float32)], axis=0)
    loop = jnp.arange(N, dtype=jnp.int32)
    pad = N + (jnp.arange(EPAD - E2, dtype=jnp.int32) % (NPAD - N))
    src = jnp.concatenate([edge_index[0].astype(jnp.int32), loop, pad])
    dst = jnp.concatenate([edge_index[1].astype(jnp.int32), loop, pad])
    out, _ = _prop(h0p, src.reshape(NT * NB, B), dst.reshape(NT * NB, B))
    return out[:N]


# speedup vs baseline: 1.0180x; 1.0083x over previous
import functools

import jax
import jax.numpy as jnp
from jax import lax
from jax.experimental import pallas as pl
from jax.experimental.pallas import tpu as pltpu
from jax.experimental.pallas import tpu_sc as plsc

N = 10000
D = 64
E = 160000
E2 = E + N
NFEAT = 256
K = 10
ALPHA = 0.1

NT = 16
B = 128
NB = 85
EPT = NB * B
EPAD = NT * EPT
CH = 640
SUB = 80
NSUB = CH // SUB
NPAD = NT * CH


def _mlp_body(x_ref, w1_ref, b1_ref, w2_ref, b2_ref, o_ref):
    h = lax.dot_general(x_ref[...], w1_ref[...], (((1,), (1,)), ((), ())),
                        preferred_element_type=jnp.float32)
    h = jnp.maximum(h + b1_ref[...], 0.0)
    o = lax.dot_general(h, w2_ref[...], (((1,), (1,)), ((), ())),
                        preferred_element_type=jnp.float32)
    o_ref[...] = o + b2_ref[...]


def _mlp(x, W1, b1, W2, b2):
    R = 1000
    return pl.pallas_call(
        _mlp_body,
        grid=(N // R,),
        in_specs=[
            pl.BlockSpec((R, NFEAT), lambda i: (i, 0)),
            pl.BlockSpec((D, NFEAT), lambda i: (0, 0)),
            pl.BlockSpec((1, D), lambda i: (0, 0)),
            pl.BlockSpec((D, D), lambda i: (0, 0)),
            pl.BlockSpec((1, D), lambda i: (0, 0)),
        ],
        out_specs=pl.BlockSpec((R, D), lambda i: (i, 0)),
        out_shape=jax.ShapeDtypeStruct((N, D), jnp.float32),
    )(x, W1, b1.reshape(1, D), W2, b2.reshape(1, D))


def _prop_body(h0_hbm, src_hbm, dst_hbm, out_hbm, g_hbm,
               acc, deg, srcv, dstv,
               rows0, rows1, rows2,
               ab0, ab1, hb0, hb1, gb0, gb1, zb,
               ones, zeros1, dbuf, dinv, asc, bsc,
               gsem, ssem, rsem_a, rsem_h, wsem_g, wsem_z):
    s = lax.axis_index("s")

    pltpu.make_async_copy(src_hbm.at[pl.ds(s * NB, NB)], srcv, gsem.at[0]).start()
    pltpu.make_async_copy(dst_hbm.at[pl.ds(s * NB, NB)], dstv, gsem.at[1]).start()

    def _initv(t, _):
        sl = pl.ds(t * 16, 16)
        ones[sl] = jnp.ones((16,), jnp.float32)
        zeros1[sl] = jnp.zeros((16,), jnp.float32)
        dbuf[sl] = jnp.ones((16,), jnp.float32)
        return 0
    lax.fori_loop(0, CH // 16, _initv, 0)

    def _initz(i, _):
        for q in range(D // 16):
            zb[i, pl.ds(q * 16, 16)] = jnp.zeros((16,), jnp.float32)
        return 0
    lax.fori_loop(0, SUB, _initz, 0)

    pltpu.sync_copy(zeros1, deg.at[pl.ds(s * CH, CH)])
    pltpu.make_async_copy(src_hbm.at[pl.ds(s * NB, NB)], srcv, gsem.at[0]).wait()
    pltpu.make_async_copy(dst_hbm.at[pl.ds(s * NB, NB)], dstv, gsem.at[1]).wait()
    plsc.subcore_barrier()

    def _degb(j, _):
        @pl.when(j >= 4)
        def _():
            pltpu.make_async_copy(ones.at[pl.ds(0, B)], deg.at[dstv.at[j - 4]],
                                  ssem.at[j % 4]).wait()
        pltpu.make_async_copy(ones.at[pl.ds(0, B)], deg.at[dstv.at[j]],
                              ssem.at[j % 4]).start(add=True)
        return 0
    lax.fori_loop(0, NB, _degb, 0)
    for r in range(4):
        j = NB - 4 + r
        pltpu.make_async_copy(ones.at[pl.ds(0, B)], deg.at[dstv.at[j]],
                              ssem.at[j % 4]).wait()
    plsc.subcore_barrier()

    pltpu.sync_copy(deg.at[pl.ds(s * CH, CH)], dbuf)

    def _newton(t, _):
        sl = pl.ds(t * 16, 16)
        x = dbuf[sl]
        xi = lax.bitcast_convert_type(x, jnp.int32)
        yi = jnp.int32(0x5F3759DF) - lax.shift_right_logical(xi, 1)
        y = lax.bitcast_convert_type(yi, jnp.float32)
        for _ in range(3):
            y = y * (1.5 - 0.5 * x * y * y)
        dinv[sl] = y
        asc[sl] = (1.0 - ALPHA) * y * y
        bsc[sl] = ALPHA * y
        return 0
    lax.fori_loop(0, CH // 16, _newton, 0)

    def _g0sub(c5, _):
        @pl.when(s * CH + c5 * SUB < N)
        def _():
            pltpu.sync_copy(h0_hbm.at[pl.ds(s * CH + c5 * SUB, SUB)], hb0)

        def _g0grp(gi, _):
            lg = c5 * SUB + gi * 16
            d16 = dinv[pl.ds(lg, 16)]
            for i in range(16):
                dv = d16[i]
                for q in range(D // 16):
                    sl = pl.ds(q * 16, 16)
                    gb0[gi * 16 + i, sl] = dv * hb0[gi * 16 + i, sl]
            return 0
        lax.fori_loop(0, SUB // 16, _g0grp, 0)
        pltpu.sync_copy(gb0, g_hbm.at[pl.ds(s * CH + c5 * SUB, SUB)])
        pltpu.sync_copy(zb, acc.at[pl.ds(s * CH + c5 * SUB, SUB)])
        return 0
    lax.fori_loop(0, NSUB, _g0sub, 0)
    plsc.subcore_barrier()

    rbufs = (rows0, rows1, rows2)

    def _estep(j, b, guarded):
        b2 = (b + 1) % 3

        def _w():
            pltpu.make_async_copy(rbufs[b2], acc.at[dstv.at[j - 2]],
                                  ssem.at[b2]).wait()

        def _g():
            pltpu.make_async_copy(g_hbm.at[srcv.at[j + 1]], rbufs[b2],
                                  gsem.at[b2]).start()
        if guarded:
            pl.when(j >= 2)(_w)
            _g()
        else:
            _w()
        pltpu.make_async_copy(g_hbm.at[srcv.at[j]], rbufs[b],
                              gsem.at[b]).wait()
        pltpu.make_async_copy(rbufs[b], acc.at[dstv.at[j]],
                              ssem.at[b]).start(add=True)

    def _edge_phase():
        pltpu.make_async_copy(g_hbm.at[srcv.at[0]], rbufs[0], gsem.at[0]).start()

        def _eb(t, _):
            for b in range(3):
                _estep(t * 3 + b, b, True)
            return 0
        lax.fori_loop(0, (NB - 1) // 3, _eb, 0)
        _estep(NB - 1, (NB - 1) % 3, False)
        for j in (NB - 2, NB - 1):
            pltpu.make_async_copy(rbufs[j % 3], acc.at[dstv.at[j]],
                                  ssem.at[j % 3]).wait()

    abufs = (ab0, ab1)
    hbufs = (hb0, hb1)
    gbufs = (gb0, gb1)

    def _u_reads(c5, p):
        n0 = s * CH + c5 * SUB
        pltpu.make_async_copy(acc.at[pl.ds(n0, SUB)], abufs[p], rsem_a.at[p]).start()

        @pl.when(n0 < N)
        def _():
            pltpu.make_async_copy(h0_hbm.at[pl.ds(n0, SUB)], hbufs[p],
                                  rsem_h.at[p]).start()

    def _u_step(c5, p, last):
        n0 = s * CH + c5 * SUB
        ab, hb, gb = abufs[p], hbufs[p], gbufs[p]

        @pl.when(c5 + 1 < NSUB)
        def _():
            _u_reads(c5 + 1, 1 - p)

        pltpu.make_async_copy(acc.at[pl.ds(n0, SUB)], ab, rsem_a.at[p]).wait()

        @pl.when(n0 < N)
        def _():
            pltpu.make_async_copy(h0_hbm.at[pl.ds(n0, SUB)], hb,
                                  rsem_h.at[p]).wait()

        nprev = n0 - 2 * SUB
        wcond = (c5 >= 2) & (nprev < N) if last else (c5 >= 2)

        @pl.when(wcond)
        def _():
            dprev = out_hbm if last else g_hbm
            pltpu.make_async_copy(gb, dprev.at[pl.ds(nprev, SUB)], wsem_g.at[p]).wait()
            pltpu.make_async_copy(zb, acc.at[pl.ds(nprev, SUB)], wsem_z.at[p]).wait()

        def _ugrp(gi, _):
            lg = c5 * SUB + gi * 16
            if last:
                a16 = (1.0 - ALPHA) * dinv[pl.ds(lg, 16)]
                b16 = jnp.full((16,), ALPHA, jnp.float32)
            else:
                a16 = asc[pl.ds(lg, 16)]
                b16 = bsc[pl.ds(lg, 16)]
            for i in range(16):
                a_s = a16[i]
                b_s = b16[i]
                r = gi * 16 + i
                for q in range(D // 16):
                    sl = pl.ds(q * 16, 16)
                    gb[r, sl] = a_s * ab[r, sl] + b_s * hb[r, sl]
            return 0
        lax.fori_loop(0, SUB // 16, _ugrp, 0)
        wcond2 = (n0 < N) if last else (n0 >= 0)

        @pl.when(wcond2)
        def _():
            dref = out_hbm if last else g_hbm
            pltpu.make_async_copy(gb, dref.at[pl.ds(n0, SUB)], wsem_g.at[p]).start()
            pltpu.make_async_copy(zb, acc.at[pl.ds(n0, SUB)], wsem_z.at[p]).start()

    def _update_phase(last):
        _u_reads(0, 0)

        def _ut(t, _):
            for pb in range(2):
                _u_step(t * 2 + pb, pb, last)
            return 0
        lax.fori_loop(0, NSUB // 2, _ut, 0)
        dref = out_hbm if last else g_hbm
        for c5 in (NSUB - 2, NSUB - 1):
            p = c5 % 2
            n0 = s * CH + c5 * SUB

            @pl.when((n0 < N) if last else (n0 >= 0))
            def _():
                pltpu.make_async_copy(gbufs[p], dref.at[pl.ds(n0, SUB)],
                                      wsem_g.at[p]).wait()
                pltpu.make_async_copy(zb, acc.at[pl.ds(n0, SUB)],
                                      wsem_z.at[p]).wait()

    def _iteration(last):
        _edge_phase()
        plsc.subcore_barrier()
        _update_phase(last)
        plsc.subcore_barrier()

    def _kb(k, _):
        _iteration(False)
        return 0
    lax.fori_loop(0, K - 1, _kb, 0)
    _iteration(True)


_prop = functools.partial(
    pl.kernel,
    out_type=(jax.ShapeDtypeStruct((N, D), jnp.float32),
              jax.ShapeDtypeStruct((NPAD, D), jnp.float32)),
    mesh=plsc.VectorSubcoreMesh(core_axis_name="c", subcore_axis_name="s",
                                num_cores=1, num_subcores=NT),
    compiler_params=pltpu.CompilerParams(use_tc_tiling_on_sc=False),
    scratch_types=[
        pltpu.VMEM_SHARED((NPAD, D), jnp.float32),
        pltpu.VMEM_SHARED((NPAD,), jnp.float32),
        pltpu.VMEM((NB, B), jnp.int32),
        pltpu.VMEM((NB, B), jnp.int32),
        pltpu.VMEM((B, D), jnp.float32),
        pltpu.VMEM((B, D), jnp.float32),
        pltpu.VMEM((B, D), jnp.float32),
        pltpu.VMEM((SUB, D), jnp.float32),
        pltpu.VMEM((SUB, D), jnp.float32),
        pltpu.VMEM((SUB, D), jnp.float32),
        pltpu.VMEM((SUB, D), jnp.float32),
        pltpu.VMEM((SUB, D), jnp.float32),
        pltpu.VMEM((SUB, D), jnp.float32),
        pltpu.VMEM((SUB, D), jnp.float32),
        pltpu.VMEM((CH,), jnp.float32),
        pltpu.VMEM((CH,), jnp.float32),
        pltpu.VMEM((CH,), jnp.float32),
        pltpu.VMEM((CH,), jnp.float32),
        pltpu.VMEM((CH,), jnp.float32),
        pltpu.VMEM((CH,), jnp.float32),
        pltpu.SemaphoreType.DMA((3,)),
        pltpu.SemaphoreType.DMA((4,)),
        pltpu.SemaphoreType.DMA((2,)),
        pltpu.SemaphoreType.DMA((2,)),
        pltpu.SemaphoreType.DMA((2,)),
        pltpu.SemaphoreType.DMA((2,)),
    ],
)(_prop_body)


def kernel(x, edge_index, W1, b1, W2, b2):
    h0 = _mlp(x, W1, b1, W2, b2)
    loop = jnp.arange(N, dtype=jnp.int32)
    pad = N + (jnp.arange(EPAD - E2, dtype=jnp.int32) % (NPAD - N))
    src = jnp.concatenate([edge_index[0].astype(jnp.int32), loop, pad])
    dst = jnp.concatenate([edge_index[1].astype(jnp.int32), loop, pad])
    out, _ = _prop(h0, src.reshape(NT * NB, B), dst.reshape(NT * NB, B))
    return out
